# baseline (speedup 1.0000x reference)
import functools

import jax
import jax.numpy as jnp
from jax import lax
from jax.experimental import pallas as pl
from jax.experimental.pallas import tpu as pltpu
from jax.experimental.pallas import tpu_sc as plsc

_ANCHORS = [1.3221, 1.73145, 3.19275, 4.00944, 5.05587, 8.09892,
            9.47112, 4.84053, 11.2364, 10.0071]
_A = 5
_OBJECT_SCALE = 5.0
_SIL_THRESH = 0.6
_L = 16

_F32 = jnp.float32
_I32 = jnp.int32


def _bcast_lane(v, i):
    idx = jnp.full((_L,), i, _I32)
    dnums = lax.GatherDimensionNumbers(
        offset_dims=(), collapsed_slice_dims=(0,), start_index_map=(0,))
    return lax.gather(v, idx[:, None], dnums, slice_sizes=(1,),
                      mode=lax.GatherScatterMode.PROMISE_IN_BOUNDS)


def _sig(x):
    return 1.0 / (1.0 + jnp.exp(-x))


def _vlog(x):
    bits = lax.bitcast_convert_type(x, _I32)
    e = (bits >> 23) - 127
    mbits = (bits & _I32(0x007FFFFF)) | _I32(0x3F800000)
    m = lax.bitcast_convert_type(mbits, _F32)
    big = m > 1.41421356237
    m = jnp.where(big, m * 0.5, m)
    e = e + jnp.where(big, 1, 0)
    t = m - 1.0
    z = t * t
    p = jnp.full((_L,), 7.0376836292e-2, _F32)
    for c in (-1.1514610310e-1, 1.1676998740e-1, -1.2420140846e-1,
              1.4249322787e-1, -1.6668057665e-1, 2.0000714765e-1,
              -2.4999993993e-1, 3.3333331174e-1):
        p = p * t + _F32(c)
    y = t * z * p - 0.5 * z
    return t + y + e.astype(_F32) * _F32(0.6931471805599453)


def _build_sc_call(B, H, W):
    HW = H * W
    ROW = _A * 5 * HW
    ROWP = ((ROW + 15) // 16) * 16
    nfull = HW // _L
    offs = [j * _L for j in range(nfull)] + ([HW - _L] if HW % _L else [])
    ndup = nfull * _L - (HW - _L)
    try:
        info = plsc.get_sparse_core_info()
        NC, NS = info.num_cores, info.num_subcores
    except Exception:
        NC, NS = 2, 16
    NW = NC * NS
    BPW = B // NW
    UVD_W = 64 * B // NW

    mesh = plsc.VectorSubcoreMesh(core_axis_name="c", subcore_axis_name="s")

    @functools.partial(
        pl.kernel, mesh=mesh,
        out_type=jax.ShapeDtypeStruct((NW, _L), _F32),
        scratch_types=[
            pltpu.VMEM((ROWP,), _F32),
            pltpu.VMEM((_L,), _F32),
            pltpu.VMEM((UVD_W,), _F32),
            pltpu.VMEM((UVD_W,), _F32),
            pltpu.VMEM((_L,), _F32),
        ],
    )
    def sc_loss(pred_hbm, targ_hbm, pu_hbm, gu_hbm, out_hbm,
                pred_v, targ_v, pu_v, gu_v, out_v):
        wid = lax.axis_index("s") * NC + lax.axis_index("c")
        zero = jnp.zeros((_L,), _F32)
        acc = zero

        lanev = lax.iota(_I32, _L)
        poscs, tms, wgs, hgs = [], [], [], []
        for off in offs:
            pos = lanev + off
            if off == HW - _L and HW % _L:
                posc = jnp.where(lanev < ndup, -1, pos)
                tms.append(jnp.where(lanev < ndup, _F32(0.0), _F32(1.0)))
            else:
                posc = pos
                tms.append(None)
            poscs.append(posc)
            wgs.append(lax.rem(pos, W).astype(_F32))
            hgs.append(lax.div(pos, W).astype(_F32))

        for k in range(BPW):
            b = wid * BPW + k
            pltpu.sync_copy(pred_hbm.at[b], pred_v)
            pltpu.sync_copy(targ_hbm.at[b], targ_v)
            tv = targ_v[...]
            gxv = _bcast_lane(tv, 0) * _F32(W)
            gyv = _bcast_lane(tv, 1) * _F32(H)
            gwv = _bcast_lane(tv, 2) * _F32(W)
            ghv = _bcast_lane(tv, 3) * _F32(H)
            gxl = gxv - gwv * 0.5
            gxr = gxv + gwv * 0.5
            gyl = gyv - ghv * 0.5
            gyr = gyv + ghv * 0.5
            garea = gwv * ghv

            bestv = jnp.zeros((_L,), _I32)
            biou = None
            for a in range(_A):
                awa = _ANCHORS[2 * a]
                aha = _ANCHORS[2 * a + 1]
                uw = jnp.maximum(gwv, _F32(awa))
                uh = jnp.maximum(ghv, _F32(aha))
                cw = (gwv + _F32(awa)) - uw
                ch = (ghv + _F32(aha)) - uh
                carea = jnp.maximum(cw, 0.0) * jnp.maximum(ch, 0.0)
                uarea = (_F32(awa * aha) + garea) - carea
                au = carea / uarea
                if biou is None:
                    biou = au
                else:
                    upd = au > biou
                    bestv = jnp.where(upd, a, bestv)
                    biou = jnp.where(upd, au, biou)
            awbv = zero
            ahbv = zero
            for a in range(_A):
                hit = bestv == a
                awbv = awbv + jnp.where(hit, _F32(_ANCHORS[2 * a]), 0.0)
                ahbv = ahbv + jnp.where(hit, _F32(_ANCHORS[2 * a + 1]), 0.0)
            lwv = _vlog(gwv / awbv)
            lhv = _vlog(ghv / ahbv)
            giv = gxv.astype(_I32)
            gjv = gyv.astype(_I32)
            dxv = gxv - giv.astype(_F32)
            dyv = gyv - gjv.astype(_F32)
            pselv = gjv * W + giv

            for a in range(_A):
                awa = _F32(_ANCHORS[2 * a])
                aha = _F32(_ANCHORS[2 * a + 1])
                bm = jnp.where(bestv == a, _F32(1.0), _F32(0.0))
                pbase = a * 5 * HW
                for j, off in enumerate(offs):
                    xr = pred_v[pl.ds(pbase + off, _L)]
                    yr = pred_v[pl.ds(pbase + HW + off, _L)]
                    twv = pred_v[pl.ds(pbase + 2 * HW + off, _L)]
                    thv = pred_v[pl.ds(pbase + 3 * HW + off, _L)]
                    cr = pred_v[pl.ds(pbase + 4 * HW + off, _L)]
                    sx = _sig(xr)
                    sy = _sig(yr)
                    cf = _sig(cr)
                    bxv = sx + wgs[j]
                    byv = sy + hgs[j]
                    bwv = jnp.exp(twv) * awa
                    bhv = jnp.exp(thv) * aha
                    mx = jnp.minimum(bxv - bwv * 0.5, gxl)
                    nx = jnp.maximum(bxv + bwv * 0.5, gxr)
                    my = jnp.minimum(byv - bhv * 0.5, gyl)
                    ny = jnp.maximum(byv + bhv * 0.5, gyr)
                    cw = (bwv + gwv) - (nx - mx)
                    ch = (bhv + ghv) - (ny - my)
                    carea = jnp.maximum(cw, 0.0) * jnp.maximum(ch, 0.0)
                    uarea = (bwv * bhv + garea) - carea
                    iou = carea / uarea
                    tm = tms[j]
                    if tm is None:
                        m01 = jnp.where(iou > _SIL_THRESH, _F32(0.0),
                                        _F32(1.0))
                    else:
                        m01 = jnp.where(iou > _SIL_THRESH, _F32(0.0), tm)
                    sxc = sx - 0.5
                    syc = sy - 0.5
                    base = sxc * sxc + syc * syc + twv * twv + thv * thv
                    cfm = cf * cf * m01
                    sel = jnp.where(poscs[j] == pselv, bm, _F32(0.0))
                    ex = sx - dxv
                    ey = sy - dyv
                    ew = twv - lwv
                    eh = thv - lhv
                    ec = cf - iou
                    quad = (ex * ex + ey * ey + ew * ew + eh * eh
                            + _OBJECT_SCALE * (ec * ec))
                    corr = quad - base - cfm
                    if tm is not None:
                        base = base * tm
                    acc = acc + base + cfm + sel * corr

        pltpu.sync_copy(pu_hbm.at[pl.ds(wid * UVD_W, UVD_W)], pu_v)
        pltpu.sync_copy(gu_hbm.at[pl.ds(wid * UVD_W, UVD_W)], gu_v)
        for c in range(UVD_W // _L):
            dv = gu_v[pl.ds(c * _L, _L)] - pu_v[pl.ds(c * _L, _L)]
            acc = acc + dv * dv

        out_v[...] = acc * 0.5
        pltpu.sync_copy(out_v, out_hbm.at[wid])

    return sc_loss, ROW, ROWP


def kernel(pred, pred_uvd, target, uvd_gt, train_out):
    B, H, W = pred.shape[0], pred.shape[2], pred.shape[3]
    sc_loss, ROW, ROWP = _build_sc_call(B, H, W)
    predf = jnp.pad(pred.reshape(B, ROW), ((0, 0), (0, ROWP - ROW)))
    targp = jnp.pad(target, ((0, 0), (0, _L - target.shape[1])))
    pu = jnp.pad(pred_uvd, ((0, 0), (0, 64 - pred_uvd.shape[1]))).reshape(-1)
    gu = jnp.pad(uvd_gt, ((0, 0), (0, 64 - uvd_gt.shape[1]))).reshape(-1)
    partials = sc_loss(predf, targp, pu, gu)
    return jnp.sum(partials)


# speedup vs baseline: 1.3022x; 1.3022x over previous
import functools

import jax
import jax.numpy as jnp
from jax import lax
from jax.experimental import pallas as pl
from jax.experimental.pallas import tpu as pltpu
from jax.experimental.pallas import tpu_sc as plsc

_ANCHORS = [1.3221, 1.73145, 3.19275, 4.00944, 5.05587, 8.09892,
            9.47112, 4.84053, 11.2364, 10.0071]
_A = 5
_OBJECT_SCALE = 5.0
_SIL_THRESH = 0.6
_L = 16

_F32 = jnp.float32
_I32 = jnp.int32


def _bcast_lane(v, i):
    idx = jnp.full((_L,), i, _I32)
    dnums = lax.GatherDimensionNumbers(
        offset_dims=(), collapsed_slice_dims=(0,), start_index_map=(0,))
    return lax.gather(v, idx[:, None], dnums, slice_sizes=(1,),
                      mode=lax.GatherScatterMode.PROMISE_IN_BOUNDS)


def _sig(x):
    return 1.0 / (1.0 + jnp.exp(-x))


def _vlog(x):
    bits = lax.bitcast_convert_type(x, _I32)
    e = (bits >> 23) - 127
    mbits = (bits & _I32(0x007FFFFF)) | _I32(0x3F800000)
    m = lax.bitcast_convert_type(mbits, _F32)
    big = m > 1.41421356237
    m = jnp.where(big, m * 0.5, m)
    e = e + jnp.where(big, 1, 0)
    t = m - 1.0
    z = t * t
    p = jnp.full((_L,), 7.0376836292e-2, _F32)
    for c in (-1.1514610310e-1, 1.1676998740e-1, -1.2420140846e-1,
              1.4249322787e-1, -1.6668057665e-1, 2.0000714765e-1,
              -2.4999993993e-1, 3.3333331174e-1):
        p = p * t + _F32(c)
    y = t * z * p - 0.5 * z
    return t + y + e.astype(_F32) * _F32(0.6931471805599453)


def _build_sc_call(B, H, W):
    HW = H * W
    ROW = _A * 5 * HW
    ROWP = ((ROW + 15) // 16) * 16
    nfull = HW // _L
    offs = [j * _L for j in range(nfull)] + ([HW - _L] if HW % _L else [])
    ndup = nfull * _L - (HW - _L)
    try:
        info = plsc.get_sparse_core_info()
        NC, NS = info.num_cores, info.num_subcores
    except Exception:
        NC, NS = 2, 16
    NW = NC * NS
    BPW = B // NW
    UVD_W = 64 * B // NW

    mesh = plsc.VectorSubcoreMesh(core_axis_name="c", subcore_axis_name="s")

    NCH = len(offs)
    GRID = NCH * _L

    @functools.partial(
        pl.kernel, mesh=mesh,
        out_type=jax.ShapeDtypeStruct((NW, _L), _F32),
        scratch_types=[
            pltpu.VMEM((ROWP,), _F32),
            pltpu.VMEM((_L,), _F32),
            pltpu.VMEM((UVD_W,), _F32),
            pltpu.VMEM((UVD_W,), _F32),
            pltpu.VMEM((_L,), _F32),
            pltpu.VMEM((GRID,), _F32),
            pltpu.VMEM((GRID,), _F32),
            pltpu.VMEM((GRID,), _I32),
            pltpu.VMEM((GRID,), _F32),
        ],
    )
    def sc_loss(pred_hbm, targ_hbm, pu_hbm, gu_hbm, out_hbm,
                pred_v, targ_v, pu_v, gu_v, out_v,
                wg_v, hg_v, psc_v, vm_v):
        wid = lax.axis_index("s") * NC + lax.axis_index("c")
        zero = jnp.zeros((_L,), _F32)
        acc = zero

        lanev = lax.iota(_I32, _L)
        for j, off in enumerate(offs):
            pos = lanev + off
            if off == HW - _L and HW % _L:
                psc_v[pl.ds(j * _L, _L)] = jnp.where(lanev < ndup, -1, pos)
                vm_v[pl.ds(j * _L, _L)] = jnp.where(lanev < ndup, _F32(0.0),
                                                    _F32(1.0))
            else:
                psc_v[pl.ds(j * _L, _L)] = pos
                vm_v[pl.ds(j * _L, _L)] = jnp.full((_L,), 1.0, _F32)
            wg_v[pl.ds(j * _L, _L)] = lax.rem(pos, W).astype(_F32)
            hg_v[pl.ds(j * _L, _L)] = lax.div(pos, W).astype(_F32)

        for k in range(BPW):
            b = wid * BPW + k
            pltpu.sync_copy(pred_hbm.at[b], pred_v)
            pltpu.sync_copy(targ_hbm.at[b], targ_v)
            tv = targ_v[...]
            gxv = _bcast_lane(tv, 0) * _F32(W)
            gyv = _bcast_lane(tv, 1) * _F32(H)
            gwv = _bcast_lane(tv, 2) * _F32(W)
            ghv = _bcast_lane(tv, 3) * _F32(H)
            gxl = gxv - gwv * 0.5
            gxr = gxv + gwv * 0.5
            gyl = gyv - ghv * 0.5
            gyr = gyv + ghv * 0.5
            garea = gwv * ghv

            bestv = jnp.zeros((_L,), _I32)
            biou = None
            for a in range(_A):
                awa = _ANCHORS[2 * a]
                aha = _ANCHORS[2 * a + 1]
                uw = jnp.maximum(gwv, _F32(awa))
                uh = jnp.maximum(ghv, _F32(aha))
                cw = (gwv + _F32(awa)) - uw
                ch = (ghv + _F32(aha)) - uh
                carea = jnp.maximum(cw, 0.0) * jnp.maximum(ch, 0.0)
                uarea = (_F32(awa * aha) + garea) - carea
                au = carea / uarea
                if biou is None:
                    biou = au
                else:
                    upd = au > biou
                    bestv = jnp.where(upd, a, bestv)
                    biou = jnp.where(upd, au, biou)
            awbv = zero
            ahbv = zero
            for a in range(_A):
                hit = bestv == a
                awbv = awbv + jnp.where(hit, _F32(_ANCHORS[2 * a]), 0.0)
                ahbv = ahbv + jnp.where(hit, _F32(_ANCHORS[2 * a + 1]), 0.0)
            lwv = _vlog(gwv / awbv)
            lhv = _vlog(ghv / ahbv)
            giv = gxv.astype(_I32)
            gjv = gyv.astype(_I32)
            dxv = gxv - giv.astype(_F32)
            dyv = gyv - gjv.astype(_F32)
            pselv = gjv * W + giv

            for a in range(_A):
                awa = _F32(_ANCHORS[2 * a])
                aha = _F32(_ANCHORS[2 * a + 1])
                bm = jnp.where(bestv == a, _F32(1.0), _F32(0.0))
                pbase = a * 5 * HW

                def chunk(j, acc, awa=awa, aha=aha, bm=bm, pbase=pbase,
                          gxl=gxl, gxr=gxr, gyl=gyl, gyr=gyr, garea=garea,
                          gwv=gwv, ghv=ghv, dxv=dxv, dyv=dyv,
                          lwv=lwv, lhv=lhv, pselv=pselv):
                    go = j * _L
                    off = jnp.minimum(go, HW - _L) + pbase
                    xr = pred_v[pl.ds(off, _L)]
                    yr = pred_v[pl.ds(off + HW, _L)]
                    twv = pred_v[pl.ds(off + 2 * HW, _L)]
                    thv = pred_v[pl.ds(off + 3 * HW, _L)]
                    cr = pred_v[pl.ds(off + 4 * HW, _L)]
                    wg = wg_v[pl.ds(go, _L)]
                    hg = hg_v[pl.ds(go, _L)]
                    psc = psc_v[pl.ds(go, _L)]
                    vm = vm_v[pl.ds(go, _L)]
                    sx = _sig(xr)
                    sy = _sig(yr)
                    cf = _sig(cr)
                    bxv = sx + wg
                    byv = sy + hg
                    bwv = jnp.exp(twv) * awa
                    bhv = jnp.exp(thv) * aha
                    mx = jnp.minimum(bxv - bwv * 0.5, gxl)
                    nx = jnp.maximum(bxv + bwv * 0.5, gxr)
                    my = jnp.minimum(byv - bhv * 0.5, gyl)
                    ny = jnp.maximum(byv + bhv * 0.5, gyr)
                    cw = (bwv + gwv) - (nx - mx)
                    ch = (bhv + ghv) - (ny - my)
                    carea = jnp.maximum(cw, 0.0) * jnp.maximum(ch, 0.0)
                    uarea = (bwv * bhv + garea) - carea
                    iou = carea / uarea
                    m01 = jnp.where(iou > _SIL_THRESH, _F32(0.0), vm)
                    sxc = sx - 0.5
                    syc = sy - 0.5
                    base = sxc * sxc + syc * syc + twv * twv + thv * thv
                    cfm = cf * cf * m01
                    sel = jnp.where(psc == pselv, bm, _F32(0.0))
                    ex = sx - dxv
                    ey = sy - dyv
                    ew = twv - lwv
                    eh = thv - lhv
                    ec = cf - iou
                    quad = (ex * ex + ey * ey + ew * ew + eh * eh
                            + _OBJECT_SCALE * (ec * ec))
                    corr = quad - base - cfm
                    return acc + base * vm + cfm + sel * corr

                acc = lax.fori_loop(0, NCH, chunk, acc)

        pltpu.sync_copy(pu_hbm.at[pl.ds(wid * UVD_W, UVD_W)], pu_v)
        pltpu.sync_copy(gu_hbm.at[pl.ds(wid * UVD_W, UVD_W)], gu_v)
        for c in range(UVD_W // _L):
            dv = gu_v[pl.ds(c * _L, _L)] - pu_v[pl.ds(c * _L, _L)]
            acc = acc + dv * dv

        out_v[...] = acc * 0.5
        pltpu.sync_copy(out_v, out_hbm.at[wid])

    return sc_loss, ROW, ROWP


def kernel(pred, pred_uvd, target, uvd_gt, train_out):
    B, H, W = pred.shape[0], pred.shape[2], pred.shape[3]
    sc_loss, ROW, ROWP = _build_sc_call(B, H, W)
    predf = jnp.pad(pred.reshape(B, ROW), ((0, 0), (0, ROWP - ROW)))
    targp = jnp.pad(target, ((0, 0), (0, _L - target.shape[1])))
    pu = jnp.pad(pred_uvd, ((0, 0), (0, 64 - pred_uvd.shape[1]))).reshape(-1)
    gu = jnp.pad(uvd_gt, ((0, 0), (0, 64 - uvd_gt.shape[1]))).reshape(-1)
    partials = sc_loss(predf, targp, pu, gu)
    return jnp.sum(partials)
